# baseline (speedup 1.0000x reference)
import jax
import jax.numpy as jnp
from jax.experimental import pallas as pl
from jax.experimental.pallas import tpu as pltpu

MAXLEN = 200
EMBED = 32


def _body(xn_ref, tok_ref, pos_ref, out_ref, rows_ref, sem):
    for r in range(MAXLEN):
        pltpu.make_async_copy(
            tok_ref.at[xn_ref[r]], rows_ref.at[r], sem
        ).start()
    pltpu.make_async_copy(
        tok_ref.at[pl.ds(0, MAXLEN), :], rows_ref, sem
    ).wait()
    out_ref[...] = rows_ref[...] + pos_ref[...]


def kernel(x, token_table, pos_table):
    xn = x[0]
    return pl.pallas_call(
        _body,
        out_shape=jax.ShapeDtypeStruct((MAXLEN, EMBED), jnp.float32),
        in_specs=[
            pl.BlockSpec(memory_space=pltpu.SMEM),
            pl.BlockSpec(memory_space=pl.ANY),
            pl.BlockSpec(memory_space=pltpu.VMEM),
        ],
        out_specs=pl.BlockSpec(memory_space=pltpu.VMEM),
        scratch_shapes=[
            pltpu.VMEM((MAXLEN, EMBED), jnp.float32),
            pltpu.SemaphoreType.DMA,
        ],
    )(xn, token_table, pos_table)


# speedup vs baseline: 1.0017x; 1.0017x over previous
import jax
import jax.numpy as jnp
from jax.experimental import pallas as pl
from jax.experimental.pallas import tpu as pltpu

MAXLEN = 200
EMBED = 32


def _body(xn_ref, tok_ref, pos_ref, out_ref, rows_ref, sem):
    for r in range(MAXLEN):
        pltpu.make_async_copy(
            tok_ref.at[r], rows_ref.at[r], sem
        ).start()
    pltpu.make_async_copy(
        tok_ref.at[pl.ds(0, MAXLEN), :], rows_ref, sem
    ).wait()
    out_ref[...] = rows_ref[...] + pos_ref[...]


def kernel(x, token_table, pos_table):
    xn = x[0]
    return pl.pallas_call(
        _body,
        out_shape=jax.ShapeDtypeStruct((MAXLEN, EMBED), jnp.float32),
        in_specs=[
            pl.BlockSpec(memory_space=pltpu.SMEM),
            pl.BlockSpec(memory_space=pl.ANY),
            pl.BlockSpec(memory_space=pltpu.VMEM),
        ],
        out_specs=pl.BlockSpec(memory_space=pltpu.VMEM),
        scratch_shapes=[
            pltpu.VMEM((MAXLEN, EMBED), jnp.float32),
            pltpu.SemaphoreType.DMA,
        ],
    )(xn, token_table, pos_table)


# speedup vs baseline: 13.2027x; 13.1799x over previous
import jax
import jax.numpy as jnp
from jax import lax
from jax.experimental import pallas as pl
from jax.experimental.pallas import tpu as pltpu

MAXLEN = 200
EMBED = 32
LW = 128


def _body(xn_smem, xn_ref, tokT_ref, pos3_ref, out3_ref, cols_ref, sem):
    copies = []
    for r in range(MAXLEN):
        base = pl.multiple_of((xn_smem[r] >> 7) * LW, LW)
        c = pltpu.make_async_copy(
            tokT_ref.at[:, pl.ds(base, LW)], cols_ref.at[r], sem
        )
        c.start()
        copies.append(c)

    lane = lax.rem(xn_ref[...], jnp.full((MAXLEN, 1), LW, jnp.int32))
    oh = (
        lax.broadcasted_iota(jnp.int32, (MAXLEN, LW), 1) == lane
    ).astype(jnp.float32)

    for c in copies:
        c.wait()
    for e in range(EMBED):
        blk = cols_ref[:, e, :]
        red = jnp.sum(blk * oh, axis=1, keepdims=True)
        out3_ref[:, e, :] = red + pos3_ref[:, e, :]


def kernel(x, token_table, pos_table):
    xn = x[0]
    tokT = token_table.T
    pos3 = pos_table.reshape(MAXLEN, EMBED, 1)
    out3 = pl.pallas_call(
        _body,
        out_shape=jax.ShapeDtypeStruct((MAXLEN, EMBED, 1), jnp.float32),
        in_specs=[
            pl.BlockSpec(memory_space=pltpu.SMEM),
            pl.BlockSpec(memory_space=pltpu.VMEM),
            pl.BlockSpec(memory_space=pl.ANY),
            pl.BlockSpec(memory_space=pltpu.VMEM),
        ],
        out_specs=pl.BlockSpec(memory_space=pltpu.VMEM),
        scratch_shapes=[
            pltpu.VMEM((MAXLEN, EMBED, LW), jnp.float32),
            pltpu.SemaphoreType.DMA,
        ],
    )(xn, xn.reshape(MAXLEN, 1), tokT, pos3)
    return out3.reshape(MAXLEN, EMBED)


# speedup vs baseline: 22.4240x; 1.6984x over previous
import jax
import jax.numpy as jnp
from jax import lax
from jax.experimental import pallas as pl
from jax.experimental.pallas import tpu as pltpu

MAXLEN = 200
EMBED = 32
LW = 128


def _body(xn_smem, xn_ref, tokT_ref, pos_ref, out_ref, cols_ref, sem):
    copies = []
    for r in range(MAXLEN):
        base = pl.multiple_of((xn_smem[r] >> 7) * LW, LW)
        c = pltpu.make_async_copy(
            tokT_ref.at[:, pl.ds(base, LW)], cols_ref.at[r], sem
        )
        c.start()
        copies.append(c)

    lane = lax.rem(xn_ref[...], jnp.full((MAXLEN, 1), LW, jnp.int32))
    oh = (
        lax.broadcasted_iota(jnp.int32, (MAXLEN, LW), 1) == lane
    ).astype(jnp.float32)

    for c in copies:
        c.wait()
    reds = []
    for e in range(EMBED):
        blk = cols_ref[:, e, :]
        reds.append(jnp.sum(blk * oh, axis=1, keepdims=True))
    out_ref[...] = jnp.concatenate(reds, axis=1) + pos_ref[...]


def kernel(x, token_table, pos_table):
    xn = x[0]
    tokT = token_table.T
    return pl.pallas_call(
        _body,
        out_shape=jax.ShapeDtypeStruct((MAXLEN, EMBED), jnp.float32),
        in_specs=[
            pl.BlockSpec(memory_space=pltpu.SMEM),
            pl.BlockSpec(memory_space=pltpu.VMEM),
            pl.BlockSpec(memory_space=pl.ANY),
            pl.BlockSpec(memory_space=pltpu.VMEM),
        ],
        out_specs=pl.BlockSpec(memory_space=pltpu.VMEM),
        scratch_shapes=[
            pltpu.VMEM((MAXLEN, EMBED, LW), jnp.float32),
            pltpu.SemaphoreType.DMA,
        ],
    )(xn, xn.reshape(MAXLEN, 1), tokT, pos_table)


# speedup vs baseline: 31.8034x; 1.4183x over previous
import jax
import jax.numpy as jnp
from jax import lax
from jax.experimental import pallas as pl
from jax.experimental.pallas import tpu as pltpu

MAXLEN = 200
EMBED = 32
LW = 128
HALF = MAXLEN // 2


def _body(xn_smem, xn_ref, tokT_ref, posT_ref, outT_ref, cols_ref, sem0, sem1):
    sems = (sem0, sem1)
    copies = [[], []]
    for h in range(2):
        for r in range(h * HALF, (h + 1) * HALF):
            base = pl.multiple_of((xn_smem[r] >> 7) * LW, LW)
            c = pltpu.make_async_copy(
                tokT_ref.at[:, pl.ds(base, LW)], cols_ref.at[r], sems[h]
            )
            c.start()
            copies[h].append(c)

    lane = lax.rem(xn_ref[...], jnp.full((MAXLEN, 1), LW, jnp.int32))
    oh = (
        lax.broadcasted_iota(jnp.int32, (MAXLEN, LW), 1) == lane
    ).astype(jnp.float32)

    reds = []
    for h in range(2):
        sl = pl.ds(h * HALF, HALF)
        for c in copies[h]:
            c.wait()
        oh_h = oh[h * HALF : (h + 1) * HALF, :]
        for e in range(EMBED):
            blk = cols_ref[sl, e, :]
            reds.append(jnp.sum(blk * oh_h, axis=1, keepdims=True))
    lo = jnp.concatenate(reds[:EMBED], axis=1)
    hi = jnp.concatenate(reds[EMBED:], axis=1)
    res = jnp.concatenate([lo, hi], axis=0)
    outT_ref[...] = res.T + posT_ref[...]


def kernel(x, token_table, pos_table):
    xn = x[0]
    tokT = token_table.T
    posT = pos_table.T
    outT = pl.pallas_call(
        _body,
        out_shape=jax.ShapeDtypeStruct((EMBED, MAXLEN), jnp.float32),
        in_specs=[
            pl.BlockSpec(memory_space=pltpu.SMEM),
            pl.BlockSpec(memory_space=pltpu.VMEM),
            pl.BlockSpec(memory_space=pl.ANY),
            pl.BlockSpec(memory_space=pltpu.VMEM),
        ],
        out_specs=pl.BlockSpec(memory_space=pltpu.VMEM),
        scratch_shapes=[
            pltpu.VMEM((MAXLEN, EMBED, LW), jnp.float32),
            pltpu.SemaphoreType.DMA,
            pltpu.SemaphoreType.DMA,
        ],
    )(xn, xn.reshape(MAXLEN, 1), tokT, posT)
    return outT.T
